# baseline (speedup 1.0000x reference)
import jax
import jax.numpy as jnp
from jax.experimental import pallas as pl
from jax.experimental.pallas import tpu as pltpu

_NUM_CLASSES = 80
_MAX_DET = 100
_IOU_THR = 0.5
_N = 20000
_ROWS_PER_GROUP = 16
_G = _N // _ROWS_PER_GROUP
_GROUP_FLAT = _ROWS_PER_GROUP * _NUM_CLASSES
_LANES = 128
_GH = 50
_GW = 25

_BIG_I32 = 2 ** 30


def _fcos_kernel(cls_ref, small_ref, boxes_out, scores_out, classes_out,
                 fused_ref, iou_ref, m_ref):
    fused = jax.nn.sigmoid(cls_ref[...]) * jax.nn.sigmoid(small_ref[:, 0:1])
    fused_ref[...] = fused
    f3 = fused.reshape(_G, _ROWS_PER_GROUP, _NUM_CLASSES)
    gmax = jnp.max(jnp.max(f3, axis=2), axis=1)
    gmax2d = gmax.reshape(_GH, _GW)
    m_ref[...] = gmax2d
    M20 = jnp.max(gmax2d, axis=1).reshape(1, _GH)

    lane_h = jax.lax.broadcasted_iota(jnp.int32, (1, _GH), 1)
    lane_w = jax.lax.broadcasted_iota(jnp.int32, (1, _GW), 1)
    lane = jax.lax.broadcasted_iota(jnp.int32, (1, _LANES), 1)
    lane_c = jax.lax.broadcasted_iota(jnp.int32, (1, _NUM_CLASSES), 1)
    blk_flat_iota = (
        jax.lax.broadcasted_iota(jnp.int32, (_ROWS_PER_GROUP, _NUM_CLASSES), 0)
        * _NUM_CLASSES
        + jax.lax.broadcasted_iota(jnp.int32, (_ROWS_PER_GROUP, _NUM_CLASSES), 1)
    )

    def topk_body(k, carry):
        M2, scv, x1v, y1v, x2v, y2v, clsv = carry
        mm = jnp.max(M2, axis=1, keepdims=True)
        h = jnp.min(jnp.where(M2 == mm, lane_h, _BIG_I32))
        mrow = m_ref[pl.ds(h, 1), :]
        sub = jnp.min(jnp.where(mrow == mm, lane_w, _BIG_I32))
        g = h * _GW + sub
        blk = fused_ref[pl.ds(g * _ROWS_PER_GROUP, _ROWS_PER_GROUP), :]
        lidx = jnp.min(jnp.where(blk == mm, blk_flat_iota, _BIG_I32))
        flat = g * _GROUP_FLAT + lidx
        blk2 = jnp.where(blk_flat_iota == lidx, -jnp.inf, blk)
        fused_ref[pl.ds(g * _ROWS_PER_GROUP, _ROWS_PER_GROUP), :] = blk2
        bm = jnp.max(jnp.max(blk2, axis=0, keepdims=True), axis=1, keepdims=True)
        mrow2 = jnp.where(lane_w == sub, bm, mrow)
        m_ref[pl.ds(h, 1), :] = mrow2
        M2 = jnp.where(lane_h == h,
                       jnp.max(mrow2, axis=1, keepdims=True), M2)
        i = flat // _NUM_CLASSES
        ck = flat % _NUM_CLASSES
        r2 = i // _NUM_CLASSES
        c2 = i % _NUM_CLASSES
        qrow = cls_ref[pl.ds(r2, 1), :]
        a = jnp.max(jnp.where(lane_c == c2, qrow, -jnp.inf), axis=1, keepdims=True)
        b = small_ref[pl.ds(r2, 1), 0:1]
        q = jnp.sqrt(jax.nn.sigmoid(a) * jax.nn.sigmoid(b))
        row = small_ref[pl.ds(i, 1), :]
        px = row[0:1, 5:6]
        py = row[0:1, 6:7]
        s = row[0:1, 7:8]
        sel_k = lane == k
        scv = jnp.where(sel_k, q, scv)
        x1v = jnp.where(sel_k, px - row[0:1, 1:2] * s, x1v)
        y1v = jnp.where(sel_k, py - row[0:1, 2:3] * s, y1v)
        x2v = jnp.where(sel_k, px + row[0:1, 3:4] * s, x2v)
        y2v = jnp.where(sel_k, py + row[0:1, 4:5] * s, y2v)
        clsv = jnp.where(sel_k, ck, clsv)
        return M2, scv, x1v, y1v, x2v, y2v, clsv

    zeros = jnp.zeros((1, _LANES), dtype=jnp.float32)
    izeros = jnp.zeros((1, _LANES), dtype=jnp.int32)
    scv0 = jnp.full((1, _LANES), -jnp.inf, dtype=jnp.float32)
    _, scv, x1v, y1v, x2v, y2v, clsv = jax.lax.fori_loop(
        0, _MAX_DET, topk_body,
        (M20, scv0, zeros, zeros, zeros, zeros, izeros))

    sub_col = jax.lax.broadcasted_iota(jnp.int32, (_LANES, 1), 0)
    lane_f = lane.astype(jnp.float32)

    def to_col(rowvec):
        w = jnp.where(lane == sub_col, rowvec, -jnp.inf)
        return jnp.max(w, axis=1, keepdims=True)

    s_col = to_col(scv)
    pre = (s_col > scv) | ((s_col == scv) & (sub_col < lane))
    rank = jnp.sum(pre.astype(jnp.float32), axis=0, keepdims=True)
    rank_col = to_col(rank)
    perm_t = (rank_col == lane_f).astype(jnp.float32)

    feats = jnp.concatenate(
        [x1v, y1v, x2v, y2v, jnp.maximum(scv, 0.0), clsv.astype(jnp.float32),
         jnp.zeros((2, _LANES), dtype=jnp.float32)], axis=0)
    sorted_f = jnp.dot(feats, perm_t,
                       precision=jax.lax.Precision.HIGHEST,
                       preferred_element_type=jnp.float32)
    x1s = sorted_f[0:1, :]
    y1s = sorted_f[1:2, :]
    x2s = sorted_f[2:3, :]
    y2s = sorted_f[3:4, :]
    qs = sorted_f[4:5, :]
    clss = sorted_f[5:6, :].astype(jnp.int32)

    area = jnp.maximum(x2s - x1s, 0.0) * jnp.maximum(y2s - y1s, 0.0)
    x1c = to_col(x1s)
    y1c = to_col(y1s)
    x2c = to_col(x2s)
    y2c = to_col(y2s)
    area_c = to_col(area)
    iw = jnp.maximum(jnp.minimum(x2c, x2s) - jnp.maximum(x1c, x1s), 0.0)
    ih = jnp.maximum(jnp.minimum(y2c, y2s) - jnp.maximum(y1c, y1s), 0.0)
    inter = iw * ih
    union = area_c + area - inter
    iou_ref[...] = jnp.where(union > 0.0, inter / union, 0.0)

    def nms_body(i, keepf):
        irow = iou_ref[pl.ds(i, 1), :]
        keep_i = jnp.max(jnp.where(lane == i, keepf, 0.0), axis=1, keepdims=True) > 0.0
        suppress = keep_i & (irow > _IOU_THR) & (lane > i)
        return jnp.where(suppress, 0.0, keepf)

    keepf = jax.lax.fori_loop(0, _MAX_DET, nms_body,
                              jnp.ones((1, _LANES), dtype=jnp.float32))
    keepv = keepf > 0.0

    boxes_out[0:1, :] = jnp.where(keepv, x1s, 0.0)
    boxes_out[1:2, :] = jnp.where(keepv, y1s, 0.0)
    boxes_out[2:3, :] = jnp.where(keepv, x2s, 0.0)
    boxes_out[3:4, :] = jnp.where(keepv, y2s, 0.0)
    boxes_out[4:8, :] = jnp.zeros((4, _LANES), dtype=jnp.float32)
    scores_out[...] = jnp.where(keepv, qs, 0.0)
    classes_out[...] = jnp.where(keepv, clss, -1)


@jax.jit
def kernel(class_preds, box_preds, centerness_preds, points, strides):
    small = jnp.concatenate(
        [centerness_preds[0], box_preds[0], points, strides],
        axis=1)

    boxes_r, scores_r, classes_r = pl.pallas_call(
        _fcos_kernel,
        out_shape=[
            jax.ShapeDtypeStruct((8, _LANES), jnp.float32),
            jax.ShapeDtypeStruct((1, _LANES), jnp.float32),
            jax.ShapeDtypeStruct((1, _LANES), jnp.int32),
        ],
        scratch_shapes=[
            pltpu.VMEM((_N, _NUM_CLASSES), jnp.float32),
            pltpu.VMEM((_LANES, _LANES), jnp.float32),
            pltpu.VMEM((_GH, _GW), jnp.float32),
        ],
    )(class_preds[0], small)

    boxes_out = boxes_r[:4, :_MAX_DET].T
    scores_out = scores_r[0, :_MAX_DET]
    classes_out = classes_r[0, :_MAX_DET]
    return boxes_out, scores_out, classes_out


# speedup vs baseline: 1.1197x; 1.1197x over previous
import jax
import jax.numpy as jnp
from jax.experimental import pallas as pl
from jax.experimental.pallas import tpu as pltpu

_NUM_CLASSES = 80
_MAX_DET = 100
_IOU_THR = 0.5
_N = 20000
_ROWS_PER_GROUP = 16
_G = _N // _ROWS_PER_GROUP
_GROUP_FLAT = _ROWS_PER_GROUP * _NUM_CLASSES
_LANES = 128

_BIG_I32 = 2 ** 30


def _fcos_kernel(cls_ref, small_ref, boxes_out, scores_out, classes_out,
                 fused_ref, iou_ref):
    fused = jax.nn.sigmoid(cls_ref[...]) * jax.nn.sigmoid(small_ref[:, 0:1])
    fused_ref[...] = fused
    f3 = fused.reshape(_G, _ROWS_PER_GROUP, _NUM_CLASSES)
    gmax = jnp.max(jnp.max(f3, axis=2), axis=1)
    M0 = gmax.reshape(1, _G)

    lane_g = jax.lax.broadcasted_iota(jnp.int32, (1, _G), 1)
    lane = jax.lax.broadcasted_iota(jnp.int32, (1, _LANES), 1)
    lane_c = jax.lax.broadcasted_iota(jnp.int32, (1, _NUM_CLASSES), 1)
    blk_flat_iota = (
        jax.lax.broadcasted_iota(jnp.int32, (_ROWS_PER_GROUP, _NUM_CLASSES), 0)
        * _NUM_CLASSES
        + jax.lax.broadcasted_iota(jnp.int32, (_ROWS_PER_GROUP, _NUM_CLASSES), 1)
    )

    def topk_body(k, carry):
        M, scv, x1v, y1v, x2v, y2v, clsv = carry
        mm = jnp.max(M, axis=1, keepdims=True)
        g = jnp.min(jnp.where(M == mm, lane_g, _BIG_I32))
        blk = fused_ref[pl.ds(g * _ROWS_PER_GROUP, _ROWS_PER_GROUP), :]
        lidx = jnp.min(jnp.where(blk == mm, blk_flat_iota, _BIG_I32))
        flat = g * _GROUP_FLAT + lidx
        blk2 = jnp.where(blk_flat_iota == lidx, -jnp.inf, blk)
        fused_ref[pl.ds(g * _ROWS_PER_GROUP, _ROWS_PER_GROUP), :] = blk2
        bm = jnp.max(jnp.max(blk2, axis=0, keepdims=True), axis=1, keepdims=True)
        M = jnp.where(lane_g == g, bm, M)
        i = flat // _NUM_CLASSES
        ck = flat % _NUM_CLASSES
        r2 = i // _NUM_CLASSES
        c2 = i % _NUM_CLASSES
        qrow = cls_ref[pl.ds(r2, 1), :]
        a = jnp.max(jnp.where(lane_c == c2, qrow, -jnp.inf), axis=1, keepdims=True)
        b = small_ref[pl.ds(r2, 1), 0:1]
        q = jnp.sqrt(jax.nn.sigmoid(a) * jax.nn.sigmoid(b))
        row = small_ref[pl.ds(i, 1), :]
        px = row[0:1, 5:6]
        py = row[0:1, 6:7]
        s = row[0:1, 7:8]
        sel_k = lane == k
        scv = jnp.where(sel_k, q, scv)
        x1v = jnp.where(sel_k, px - row[0:1, 1:2] * s, x1v)
        y1v = jnp.where(sel_k, py - row[0:1, 2:3] * s, y1v)
        x2v = jnp.where(sel_k, px + row[0:1, 3:4] * s, x2v)
        y2v = jnp.where(sel_k, py + row[0:1, 4:5] * s, y2v)
        clsv = jnp.where(sel_k, ck, clsv)
        return M, scv, x1v, y1v, x2v, y2v, clsv

    zeros = jnp.zeros((1, _LANES), dtype=jnp.float32)
    izeros = jnp.zeros((1, _LANES), dtype=jnp.int32)
    scv0 = jnp.full((1, _LANES), -jnp.inf, dtype=jnp.float32)
    _, scv, x1v, y1v, x2v, y2v, clsv = jax.lax.fori_loop(
        0, _MAX_DET, topk_body,
        (M0, scv0, zeros, zeros, zeros, zeros, izeros))

    sub_col = jax.lax.broadcasted_iota(jnp.int32, (_LANES, 1), 0)
    lane_f = lane.astype(jnp.float32)

    def to_col(rowvec):
        w = jnp.where(lane == sub_col, rowvec, -jnp.inf)
        return jnp.max(w, axis=1, keepdims=True)

    s_col = to_col(scv)
    pre = (s_col > scv) | ((s_col == scv) & (sub_col < lane))
    rank = jnp.sum(pre.astype(jnp.float32), axis=0, keepdims=True)
    rank_col = to_col(rank)
    perm_t = (rank_col == lane_f).astype(jnp.float32)

    feats = jnp.concatenate(
        [x1v, y1v, x2v, y2v, jnp.maximum(scv, 0.0), clsv.astype(jnp.float32),
         jnp.zeros((2, _LANES), dtype=jnp.float32)], axis=0)
    sorted_f = jnp.dot(feats, perm_t,
                       precision=jax.lax.Precision.HIGHEST,
                       preferred_element_type=jnp.float32)
    x1s = sorted_f[0:1, :]
    y1s = sorted_f[1:2, :]
    x2s = sorted_f[2:3, :]
    y2s = sorted_f[3:4, :]
    qs = sorted_f[4:5, :]
    clss = sorted_f[5:6, :].astype(jnp.int32)

    area = jnp.maximum(x2s - x1s, 0.0) * jnp.maximum(y2s - y1s, 0.0)
    x1c = to_col(x1s)
    y1c = to_col(y1s)
    x2c = to_col(x2s)
    y2c = to_col(y2s)
    area_c = to_col(area)
    iw = jnp.maximum(jnp.minimum(x2c, x2s) - jnp.maximum(x1c, x1s), 0.0)
    ih = jnp.maximum(jnp.minimum(y2c, y2s) - jnp.maximum(y1c, y1s), 0.0)
    inter = iw * ih
    union = area_c + area - inter
    iou_ref[...] = jnp.where(union > 0.0, inter / union, 0.0)

    def nms_body(i, keepf):
        irow = iou_ref[pl.ds(i, 1), :]
        keep_i = jnp.max(jnp.where(lane == i, keepf, 0.0), axis=1, keepdims=True) > 0.0
        suppress = keep_i & (irow > _IOU_THR) & (lane > i)
        return jnp.where(suppress, 0.0, keepf)

    keepf = jax.lax.fori_loop(0, _MAX_DET, nms_body,
                              jnp.ones((1, _LANES), dtype=jnp.float32))
    keepv = keepf > 0.0

    boxes_out[0:1, :] = jnp.where(keepv, x1s, 0.0)
    boxes_out[1:2, :] = jnp.where(keepv, y1s, 0.0)
    boxes_out[2:3, :] = jnp.where(keepv, x2s, 0.0)
    boxes_out[3:4, :] = jnp.where(keepv, y2s, 0.0)
    boxes_out[4:8, :] = jnp.zeros((4, _LANES), dtype=jnp.float32)
    scores_out[...] = jnp.where(keepv, qs, 0.0)
    classes_out[...] = jnp.where(keepv, clss, -1)


@jax.jit
def kernel(class_preds, box_preds, centerness_preds, points, strides):
    small = jnp.concatenate(
        [centerness_preds[0], box_preds[0], points, strides],
        axis=1)

    boxes_r, scores_r, classes_r = pl.pallas_call(
        _fcos_kernel,
        out_shape=[
            jax.ShapeDtypeStruct((8, _LANES), jnp.float32),
            jax.ShapeDtypeStruct((1, _LANES), jnp.float32),
            jax.ShapeDtypeStruct((1, _LANES), jnp.int32),
        ],
        scratch_shapes=[
            pltpu.VMEM((_N, _NUM_CLASSES), jnp.float32),
            pltpu.VMEM((_LANES, _LANES), jnp.float32),
        ],
    )(class_preds[0], small)

    boxes_out = boxes_r[:4, :_MAX_DET].T
    scores_out = scores_r[0, :_MAX_DET]
    classes_out = classes_r[0, :_MAX_DET]
    return boxes_out, scores_out, classes_out
